# baseline (speedup 1.0000x reference)
import functools

import jax
import jax.numpy as jnp
from jax import lax
from jax.experimental import pallas as pl
from jax.experimental.pallas import tpu as pltpu
from jax.experimental.pallas import tpu_sc as plsc

NK = 512
NN = 1_000_000
SL = 31_264
NP = 32 * SL
NB = 2048
CAP = 128
NCAND = 32 * CAP
ITER = SL // 16

SL2 = 31_280
NPP = 32 * SL2
ZCH = SL2 // 5
HALF = 16 * SL2

WCAP = 544
ECAP = 2048
E2CAP = 512

IMIN = -2147483648

_MESH_CACHE = []


def _get_mesh():
    if not _MESH_CACHE:
        _MESH_CACHE.append(plsc.VectorSubcoreMesh(
            core_axis_name="c", subcore_axis_name="s"))
    return _MESH_CACHE[0]


def _key_of(v16):
    b = lax.bitcast_convert_type(v16, jnp.int32)
    sgn = lax.shift_right_arithmetic(b, 31)
    return b ^ (sgn & jnp.int32(0x7FFFFFFF))


def _val_of(k16):
    sgn = lax.shift_right_arithmetic(k16, 31)
    return lax.bitcast_convert_type(k16 ^ (sgn & jnp.int32(0x7FFFFFFF)), jnp.float32)


def _iota16():
    return lax.broadcasted_iota(jnp.int32, (16,), 0)


def _lane0(v):
    return lax.squeeze(lax.slice(v, (0,), (1,)), (0,))


def _lane15(v):
    return lax.squeeze(lax.slice(v, (15,), (16,)), (0,))


def _stage1(x_hbm, ck_hbm, ci_hbm, h2_hbm,
            data_v, lhist_v, mhist_v, ghist_v, ck_v, ci_v, cut_v, sh_v, shc_v):
    c = lax.axis_index("c")
    s = lax.axis_index("s")
    wid = c * 16 + s
    base = wid * SL
    iota = _iota16()
    zeros16 = jnp.zeros((16,), jnp.int32)
    ones16 = jnp.ones((16,), jnp.int32)

    pltpu.sync_copy(x_hbm.at[pl.ds(base, SL)], data_v)

    def zl(i, _):
        lhist_v[pl.ds(i * 16, 16)] = zeros16
        return 0
    lax.fori_loop(0, NB * 16 // 16, zl, 0, unroll=8)

    lanemul = iota * NB

    def hist_body2(i):
        v = data_v[pl.ds(i * 16, 16)]
        key = _key_of(v)
        bin1 = lax.shift_right_arithmetic(key, 21) + jnp.int32(1024)
        plsc.addupdate_scatter(lhist_v, [bin1 + lanemul], ones16)
    plsc.parallel_loop(0, ITER, 1, unroll=8)(hist_body2)

    def merge_body(j, _):
        acc = zeros16
        for l in range(16):
            acc = acc + lhist_v[pl.ds(l * NB + j * 16, 16)]
        mhist_v[pl.ds(j * 16, 16)] = acc
        return 0
    lax.fori_loop(0, NB // 16, merge_body, 0)

    pltpu.sync_copy(mhist_v, sh_v.at[s])
    plsc.subcore_barrier()

    @pl.when(s == 0)
    def _select_cutoff():
        pltpu.sync_copy(sh_v, ghist_v)

        def merge2(j, _):
            acc = zeros16
            for l in range(16):
                acc = acc + ghist_v[l, pl.ds(j * 16, 16)]
            mhist_v[pl.ds(j * 16, 16)] = acc
            return 0
        lax.fori_loop(0, NB // 16, merge2, 0)

        pltpu.sync_copy(mhist_v, h2_hbm.at[c])

        def cut_body(jj, carry):
            c_above, cmax = carry
            j = NB // 16 - 1 - jj
            chunk = mhist_v[pl.ds(j * 16, 16)]
            csum = plsc.cumsum(chunk)
            tot = _lane15(csum)
            suffix = tot - csum + chunk
            cnt = c_above + suffix
            bins = iota + j * 16
            cand = jnp.where(cnt >= NK, bins, jnp.int32(-1))
            return (c_above + tot, jnp.maximum(cmax, cand))
        _, cmax = lax.fori_loop(
            0, NB // 16, cut_body,
            (jnp.int32(0), jnp.full((16,), -1, jnp.int32)))
        cbin = jnp.max(cmax)
        cut_key = lax.shift_left(cbin - jnp.int32(1024), 21)
        cut_v[...] = jnp.full((16,), 1, jnp.int32) * cut_key
        pltpu.sync_copy(cut_v, shc_v)

    plsc.subcore_barrier()
    pltpu.sync_copy(shc_v, cut_v)
    cut_key = jnp.max(cut_v[...])

    minkey16 = jnp.full((16,), IMIN, jnp.int32)
    pad16 = jnp.full((16,), NP, jnp.int32)
    for t in range(CAP // 16):
        ck_v[pl.ds(t * 16, 16)] = minkey16
        ci_v[pl.ds(t * 16, 16)] = pad16

    def cp_body(i, cnt):
        v = data_v[pl.ds(i * 16, 16)]
        key = _key_of(v)
        m = key >= cut_key
        gidx = base + i * 16 + iota
        cc = jnp.minimum(cnt, CAP - 16)
        plsc.store_compressed(ck_v.at[pl.ds(cc, 16)], key, mask=m)
        plsc.store_compressed(ci_v.at[pl.ds(cc, 16)], gidx, mask=m)
        return cnt + _lane0(plsc.all_reduce_population_count(m))
    lax.fori_loop(0, ITER, cp_body, jnp.int32(0), unroll=8)

    pltpu.sync_copy(ck_v, ck_hbm.at[pl.ds(wid * CAP, CAP)])
    pltpu.sync_copy(ci_v, ci_hbm.at[pl.ds(wid * CAP, CAP)])


def _append(dst_k, dst_i, cnt, cap, key, gidx, m):
    cc = jnp.minimum(cnt, cap - 16)
    plsc.store_compressed(dst_k.at[pl.ds(cc, 16)], key, mask=m)
    plsc.store_compressed(dst_i.at[pl.ds(cc, 16)], gidx, mask=m)
    return cnt + _lane0(plsc.all_reduce_population_count(m))


def _stage2(ck_hbm, ci_hbm, h2_hbm, out_hbm,
            zero_v, ckv, civ, h2v, mh_v, wk_v, wi_v, ek_v, ei_v,
            lh2_v, mh2_v, e2k_v, e2i_v, si_v, sv_v, sem):
    c = lax.axis_index("c")
    s = lax.axis_index("s")
    wid = c * 16 + s
    base2 = wid * SL2
    iota = _iota16()
    zeros16f = jnp.zeros((16,), jnp.float32)
    zeros16 = jnp.zeros((16,), jnp.int32)

    def zf(i, _):
        zero_v[pl.ds(i * 16, 16)] = zeros16f
        return 0
    lax.fori_loop(0, ZCH // 16, zf, 0, unroll=8)
    for t in range(5):
        pltpu.sync_copy(zero_v, out_hbm.at[pl.ds(base2 + t * ZCH, ZCH)])

    @pl.when(s == 0)
    def _select():
        pltpu.sync_copy(ck_hbm, ckv)
        pltpu.sync_copy(ci_hbm, civ)
        pltpu.sync_copy(h2_hbm, h2v)

        def mh_body(j, _):
            mh_v[pl.ds(j * 16, 16)] = (h2v[0, pl.ds(j * 16, 16)]
                                       + h2v[1, pl.ds(j * 16, 16)])
            return 0
        lax.fori_loop(0, NB // 16, mh_body, 0)

        def cut_body(jj, carry):
            c_above, cmax = carry
            j = NB // 16 - 1 - jj
            chunk = mh_v[pl.ds(j * 16, 16)]
            csum = plsc.cumsum(chunk)
            tot = _lane15(csum)
            cnt = c_above + (tot - csum + chunk)
            bins = iota + j * 16
            cand = jnp.where(cnt >= NK, bins, jnp.int32(-1))
            return (c_above + tot, jnp.maximum(cmax, cand))
        _, cmax = lax.fori_loop(
            0, NB // 16, cut_body,
            (jnp.int32(0), jnp.full((16,), -1, jnp.int32)))
        c1 = jnp.max(cmax)

        def ge_body(j, acc):
            g_acc, e_acc = acc
            chunk = mh_v[pl.ds(j * 16, 16)]
            bins = iota + j * 16
            g_acc = g_acc + jnp.where(bins > c1, chunk, zeros16)
            e_acc = e_acc + jnp.where(bins == c1, chunk, zeros16)
            return (g_acc, e_acc)
        g_acc, e_acc = lax.fori_loop(0, NB // 16, ge_body, (zeros16, zeros16))
        need1 = NK - jnp.sum(g_acc)

        minkey16 = jnp.full((16,), IMIN, jnp.int32)
        pad16 = jnp.full((16,), NP, jnp.int32)
        for t in range(WCAP // 16):
            wk_v[pl.ds(t * 16, 16)] = minkey16
            wi_v[pl.ds(t * 16, 16)] = pad16

        def cls1(i, carry):
            cw, ce = carry
            key = ckv[pl.ds(i * 16, 16)]
            gidx = civ[pl.ds(i * 16, 16)]
            bin1 = lax.shift_right_arithmetic(key, 21) + jnp.int32(1024)
            cw = _append(wk_v, wi_v, cw, WCAP, key, gidx, bin1 > c1)
            ce = _append(ek_v, ei_v, ce, ECAP, key, gidx, bin1 == c1)
            return (cw, ce)
        cw, ce = lax.fori_loop(0, NCAND // 16, cls1,
                               (jnp.int32(0), jnp.int32(0)), unroll=2)

        def zl2(i, _):
            lh2_v[pl.ds(i * 16, 16)] = zeros16
            return 0
        lax.fori_loop(0, 256 * 16 // 16, zl2, 0, unroll=8)

        ones16 = jnp.ones((16,), jnp.int32)
        lanemul2 = iota * 256
        ne_it = ECAP // 16
        ne2_it_s = E2CAP // 16

        def h2_body(i, _):
            key = ek_v[pl.ds(i * 16, 16)]
            valid = (i * 16 + iota) < ce
            bin2 = lax.shift_right_logical(key, 13) & jnp.int32(0xFF)
            plsc.addupdate_scatter(lh2_v, [bin2 + lanemul2], ones16,
                                   mask=valid)
            return 0
        lax.fori_loop(0, ne_it, h2_body, 0)

        def merge2(j, _):
            acc = zeros16
            for l in range(16):
                acc = acc + lh2_v[pl.ds(l * 256 + j * 16, 16)]
            mh2_v[pl.ds(j * 16, 16)] = acc
            return 0
        lax.fori_loop(0, 16, merge2, 0)

        def cut2_body(jj, carry):
            c_above, cmax2 = carry
            j = 15 - jj
            chunk = mh2_v[pl.ds(j * 16, 16)]
            csum = plsc.cumsum(chunk)
            tot = jnp.sum(chunk)
            cnt = c_above + (tot - csum + chunk)
            bins = iota + j * 16
            cand = jnp.where(cnt >= need1, bins, jnp.int32(-1))
            return (c_above + tot, jnp.maximum(cmax2, cand))
        _, cmax2 = lax.fori_loop(
            0, 16, cut2_body, (jnp.int32(0), jnp.full((16,), -1, jnp.int32)))
        c2 = jnp.max(cmax2)

        def ge2_body(j, acc):
            chunk = mh2_v[pl.ds(j * 16, 16)]
            bins = iota + j * 16
            return acc + jnp.where(bins > c2, chunk, zeros16)
        g2_acc = lax.fori_loop(0, 16, ge2_body, zeros16)
        need2 = need1 - jnp.sum(g2_acc)

        def cls2(i, carry):
            cw, ce2 = carry
            key = ek_v[pl.ds(i * 16, 16)]
            gidx = ei_v[pl.ds(i * 16, 16)]
            valid = (i * 16 + iota) < ce
            bin2 = lax.shift_right_logical(key, 13) & jnp.int32(0xFF)
            cw = _append(wk_v, wi_v, cw, WCAP, key, gidx,
                         valid & (bin2 > c2))
            ce2 = _append(e2k_v, e2i_v, ce2, E2CAP, key, gidx,
                          valid & (bin2 == c2))
            return (cw, ce2)
        cw, ce2 = lax.fori_loop(0, ne_it, cls2, (cw, jnp.int32(0)))

        ne2_it = ne2_it_s

        k0 = e2k_v[pl.ds(0, 16)]
        hb = _lane0(k0) & jnp.int32(~0x1FFF)

        def vb_round(r, t):
            candk = t | lax.shift_left(jnp.int32(1), jnp.int32(12) - r)

            def cntb(i, acc):
                key = e2k_v[pl.ds(i * 16, 16)]
                valid = (i * 16 + iota) < ce2
                return acc + jnp.where(valid & (key >= candk), ones16, zeros16)
            cnt = jnp.sum(lax.fori_loop(0, ne2_it, cntb, zeros16))
            return jnp.where(cnt >= need2, candk, t)
        tkey = lax.fori_loop(0, 13, vb_round, hb)

        def gte_body(i, g):
            key = e2k_v[pl.ds(i * 16, 16)]
            valid = (i * 16 + iota) < ce2
            return g + jnp.where(valid & (key > tkey), ones16, zeros16)
        gt_cnt = jnp.sum(lax.fori_loop(0, ne2_it, gte_body, zeros16))
        need3 = need2 - gt_cnt

        def ib_round(r, t2):
            candi = t2 | lax.shift_left(jnp.int32(1), jnp.int32(19) - r)

            def cntb(i, acc):
                key = e2k_v[pl.ds(i * 16, 16)]
                gidx = e2i_v[pl.ds(i * 16, 16)]
                valid = (i * 16 + iota) < ce2
                m = valid & (key == tkey) & (gidx < candi)
                return acc + jnp.where(m, ones16, zeros16)
            cnt = jnp.sum(lax.fori_loop(0, ne2_it, cntb, zeros16))
            return jnp.where(cnt < need3, candi, t2)
        t2 = lax.fori_loop(0, 20, ib_round, jnp.int32(0))

        def cls3(i, cw):
            key = e2k_v[pl.ds(i * 16, 16)]
            gidx = e2i_v[pl.ds(i * 16, 16)]
            valid = (i * 16 + iota) < ce2
            m = valid & ((key > tkey) | ((key == tkey) & (gidx <= t2)))
            return _append(wk_v, wi_v, cw, WCAP, key, gidx, m)
        cw = lax.fori_loop(0, ne2_it, cls3, cw)

        lo = c * HALF
        for t in range(32):
            key = wk_v[pl.ds(t * 16, 16)]
            gidx = wi_v[pl.ds(t * 16, 16)]
            own = (gidx >= lo) & (gidx < lo + HALF)
            si_v[t // 8, pl.ds((t % 8) * 16, 16)] = jnp.where(
                own, gidx, jnp.full((16,), NP, jnp.int32))
            sv_v[t // 8, pl.ds((t % 8) * 16, 16)] = jnp.where(
                own, _val_of(key), zeros16f)

    plsc.subcore_barrier()

    @pl.when(s == 0)
    def _scatter():
        for j in range(4):
            pltpu.async_copy(sv_v.at[j], out_hbm.at[si_v.at[j]], sem).wait()


def _make_stage1():
  return functools.partial(
    pl.kernel,
    mesh=_get_mesh(),
    name="sc_stage1_hist_compact",
    compiler_params=pltpu.CompilerParams(needs_layout_passes=False),
    out_type=[
        jax.ShapeDtypeStruct((NCAND,), jnp.int32),
        jax.ShapeDtypeStruct((NCAND,), jnp.int32),
        jax.ShapeDtypeStruct((2, NB), jnp.int32),
    ],
    scratch_types=[
        pltpu.VMEM((SL,), jnp.float32),
        pltpu.VMEM((NB * 16,), jnp.int32),
        pltpu.VMEM((NB,), jnp.int32),
        pltpu.VMEM((16, NB), jnp.int32),
        pltpu.VMEM((CAP,), jnp.int32),
        pltpu.VMEM((CAP,), jnp.int32),
        pltpu.VMEM((16,), jnp.int32),
        pltpu.VMEM_SHARED((16, NB), jnp.int32),
        pltpu.VMEM_SHARED((16,), jnp.int32),
    ]
  )(_stage1)


def _make_stage2():
  return functools.partial(
    pl.kernel,
    mesh=_get_mesh(),
    name="sc_stage2_select_scatter",
    compiler_params=pltpu.CompilerParams(needs_layout_passes=False),
    out_type=[jax.ShapeDtypeStruct((NPP,), jnp.float32)],
    scratch_types=[
        pltpu.VMEM((ZCH,), jnp.float32),
        pltpu.VMEM((NCAND,), jnp.int32),
        pltpu.VMEM((NCAND,), jnp.int32),
        pltpu.VMEM((2, NB), jnp.int32),
        pltpu.VMEM((NB,), jnp.int32),
        pltpu.VMEM((WCAP,), jnp.int32),
        pltpu.VMEM((WCAP,), jnp.int32),
        pltpu.VMEM((ECAP,), jnp.int32),
        pltpu.VMEM((ECAP,), jnp.int32),
        pltpu.VMEM((256 * 16,), jnp.int32),
        pltpu.VMEM((256,), jnp.int32),
        pltpu.VMEM((E2CAP,), jnp.int32),
        pltpu.VMEM((E2CAP,), jnp.int32),
        pltpu.VMEM((4, 128), jnp.int32),
        pltpu.VMEM((4, 128), jnp.float32),
        pltpu.SemaphoreType.DMA,
    ],
  )(_stage2)


@jax.jit
def kernel(x):
    xp = jnp.concatenate(
        [x, jnp.full((NP - NN,), -jnp.inf, dtype=jnp.float32)])
    ck, ci, h2 = _make_stage1()(xp)
    (out,) = _make_stage2()(ck, ci, h2)
    return out[:NN]


# speedup vs baseline: 1.0258x; 1.0258x over previous
import functools

import jax
import jax.numpy as jnp
from jax import lax
from jax.experimental import pallas as pl
from jax.experimental.pallas import tpu as pltpu
from jax.experimental.pallas import tpu_sc as plsc

NK = 512
NN = 1_000_000
SL = 31_264
NP = 32 * SL
NB = 2048
CAP = 128
NCAND = 32 * CAP
ITER = SL // 16

SL2 = 31_280
NPP = 32 * SL2
ZCH = SL2 // 5
HALF = 16 * SL2

WCAP = 544
ECAP = 2048
E2CAP = 128

IMIN = -2147483648

_MESH_CACHE = []


def _get_mesh():
    if not _MESH_CACHE:
        _MESH_CACHE.append(plsc.VectorSubcoreMesh(
            core_axis_name="c", subcore_axis_name="s"))
    return _MESH_CACHE[0]


def _key_of(v16):
    b = lax.bitcast_convert_type(v16, jnp.int32)
    sgn = lax.shift_right_arithmetic(b, 31)
    return b ^ (sgn & jnp.int32(0x7FFFFFFF))


def _val_of(k16):
    sgn = lax.shift_right_arithmetic(k16, 31)
    return lax.bitcast_convert_type(k16 ^ (sgn & jnp.int32(0x7FFFFFFF)), jnp.float32)


def _iota16():
    return lax.broadcasted_iota(jnp.int32, (16,), 0)


def _lane0(v):
    return lax.squeeze(lax.slice(v, (0,), (1,)), (0,))


def _lane15(v):
    return lax.squeeze(lax.slice(v, (15,), (16,)), (0,))


def _stage1(x_hbm, ck_hbm, ci_hbm, h2_hbm,
            data_v, lhist_v, mhist_v, ghist_v, ck_v, ci_v, cut_v, sh_v, shc_v):
    c = lax.axis_index("c")
    s = lax.axis_index("s")
    wid = c * 16 + s
    base = wid * SL
    iota = _iota16()
    zeros16 = jnp.zeros((16,), jnp.int32)
    ones16 = jnp.ones((16,), jnp.int32)

    pltpu.sync_copy(x_hbm.at[pl.ds(base, SL)], data_v)

    def zl(i, _):
        lhist_v[pl.ds(i * 16, 16)] = zeros16
        return 0
    lax.fori_loop(0, NB * 16 // 16, zl, 0, unroll=8)

    lanemul = iota * NB

    def hist_body2(i):
        v = data_v[pl.ds(i * 16, 16)]
        key = _key_of(v)
        bin1 = lax.shift_right_arithmetic(key, 21) + jnp.int32(1024)
        plsc.addupdate_scatter(lhist_v, [bin1 + lanemul], ones16)
    plsc.parallel_loop(0, ITER, 1, unroll=8)(hist_body2)

    def merge_body(j, _):
        acc = zeros16
        for l in range(16):
            acc = acc + lhist_v[pl.ds(l * NB + j * 16, 16)]
        mhist_v[pl.ds(j * 16, 16)] = acc
        return 0
    lax.fori_loop(0, NB // 16, merge_body, 0)

    pltpu.sync_copy(mhist_v, sh_v.at[s])
    plsc.subcore_barrier()

    @pl.when(s == 0)
    def _select_cutoff():
        pltpu.sync_copy(sh_v, ghist_v)

        def merge2(j, _):
            acc = zeros16
            for l in range(16):
                acc = acc + ghist_v[l, pl.ds(j * 16, 16)]
            mhist_v[pl.ds(j * 16, 16)] = acc
            return 0
        lax.fori_loop(0, NB // 16, merge2, 0)

        pltpu.sync_copy(mhist_v, h2_hbm.at[c])

        def cut_body(jj, carry):
            c_above, cmax = carry
            j = NB // 16 - 1 - jj
            chunk = mhist_v[pl.ds(j * 16, 16)]
            csum = plsc.cumsum(chunk)
            tot = _lane15(csum)
            suffix = tot - csum + chunk
            cnt = c_above + suffix
            bins = iota + j * 16
            cand = jnp.where(cnt >= NK, bins, jnp.int32(-1))
            return (c_above + tot, jnp.maximum(cmax, cand))
        _, cmax = lax.fori_loop(
            0, NB // 16, cut_body,
            (jnp.int32(0), jnp.full((16,), -1, jnp.int32)))
        cbin = jnp.max(cmax)
        cut_key = lax.shift_left(cbin - jnp.int32(1024), 21)
        cut_v[...] = jnp.full((16,), 1, jnp.int32) * cut_key
        pltpu.sync_copy(cut_v, shc_v)

    plsc.subcore_barrier()
    pltpu.sync_copy(shc_v, cut_v)
    cut_key = jnp.max(cut_v[...])

    minkey16 = jnp.full((16,), IMIN, jnp.int32)
    pad16 = jnp.full((16,), NP, jnp.int32)
    for t in range(CAP // 16):
        ck_v[pl.ds(t * 16, 16)] = minkey16
        ci_v[pl.ds(t * 16, 16)] = pad16

    def cp_body(i, cnt):
        v = data_v[pl.ds(i * 16, 16)]
        key = _key_of(v)
        m = key >= cut_key
        gidx = base + i * 16 + iota
        cc = jnp.minimum(cnt, CAP - 16)
        plsc.store_compressed(ck_v.at[pl.ds(cc, 16)], key, mask=m)
        plsc.store_compressed(ci_v.at[pl.ds(cc, 16)], gidx, mask=m)
        return cnt + _lane0(plsc.all_reduce_population_count(m))
    lax.fori_loop(0, ITER, cp_body, jnp.int32(0), unroll=8)

    pltpu.sync_copy(ck_v, ck_hbm.at[pl.ds(wid * CAP, CAP)])
    pltpu.sync_copy(ci_v, ci_hbm.at[pl.ds(wid * CAP, CAP)])


def _append(dst_k, dst_i, cnt, cap, key, gidx, m):
    cc = jnp.minimum(cnt, cap - 16)
    plsc.store_compressed(dst_k.at[pl.ds(cc, 16)], key, mask=m)
    plsc.store_compressed(dst_i.at[pl.ds(cc, 16)], gidx, mask=m)
    return cnt + _lane0(plsc.all_reduce_population_count(m))


def _stage2(ck_hbm, ci_hbm, h2_hbm, out_hbm,
            zero_v, ckv, civ, h2v, mh_v, wk_v, wi_v, ek_v, ei_v,
            lh2_v, mh2_v, e2k_v, e2i_v, si_v, sv_v, sem):
    c = lax.axis_index("c")
    s = lax.axis_index("s")
    wid = c * 16 + s
    base2 = wid * SL2
    iota = _iota16()
    zeros16f = jnp.zeros((16,), jnp.float32)
    zeros16 = jnp.zeros((16,), jnp.int32)

    def zf(i, _):
        zero_v[pl.ds(i * 16, 16)] = zeros16f
        return 0
    lax.fori_loop(0, ZCH // 16, zf, 0, unroll=8)
    for t in range(5):
        pltpu.sync_copy(zero_v, out_hbm.at[pl.ds(base2 + t * ZCH, ZCH)])

    @pl.when(s == 0)
    def _select():
        pltpu.sync_copy(ck_hbm, ckv)
        pltpu.sync_copy(ci_hbm, civ)
        pltpu.sync_copy(h2_hbm, h2v)

        def mh_body(j, _):
            mh_v[pl.ds(j * 16, 16)] = (h2v[0, pl.ds(j * 16, 16)]
                                       + h2v[1, pl.ds(j * 16, 16)])
            return 0
        lax.fori_loop(0, NB // 16, mh_body, 0)

        def cut_body(jj, carry):
            c_above, cmax = carry
            j = NB // 16 - 1 - jj
            chunk = mh_v[pl.ds(j * 16, 16)]
            csum = plsc.cumsum(chunk)
            tot = _lane15(csum)
            cnt = c_above + (tot - csum + chunk)
            bins = iota + j * 16
            cand = jnp.where(cnt >= NK, bins, jnp.int32(-1))
            return (c_above + tot, jnp.maximum(cmax, cand))
        _, cmax = lax.fori_loop(
            0, NB // 16, cut_body,
            (jnp.int32(0), jnp.full((16,), -1, jnp.int32)))
        c1 = jnp.max(cmax)

        def ge_body(j, acc):
            g_acc, e_acc = acc
            chunk = mh_v[pl.ds(j * 16, 16)]
            bins = iota + j * 16
            g_acc = g_acc + jnp.where(bins > c1, chunk, zeros16)
            e_acc = e_acc + jnp.where(bins == c1, chunk, zeros16)
            return (g_acc, e_acc)
        g_acc, e_acc = lax.fori_loop(0, NB // 16, ge_body, (zeros16, zeros16))
        need1 = NK - jnp.sum(g_acc)

        minkey16 = jnp.full((16,), IMIN, jnp.int32)
        pad16 = jnp.full((16,), NP, jnp.int32)
        for t in range(WCAP // 16):
            wk_v[pl.ds(t * 16, 16)] = minkey16
            wi_v[pl.ds(t * 16, 16)] = pad16

        def cls1(i, carry):
            cw, ce = carry
            key = ckv[pl.ds(i * 16, 16)]
            gidx = civ[pl.ds(i * 16, 16)]
            bin1 = lax.shift_right_arithmetic(key, 21) + jnp.int32(1024)
            cw = _append(wk_v, wi_v, cw, WCAP, key, gidx, bin1 > c1)
            ce = _append(ek_v, ei_v, ce, ECAP, key, gidx, bin1 == c1)
            return (cw, ce)
        cw, ce = lax.fori_loop(0, NCAND // 16, cls1,
                               (jnp.int32(0), jnp.int32(0)), unroll=2)

        def zl2(i, _):
            lh2_v[pl.ds(i * 16, 16)] = zeros16
            return 0
        lax.fori_loop(0, 256 * 16 // 16, zl2, 0, unroll=8)

        ones16 = jnp.ones((16,), jnp.int32)
        lanemul2 = iota * 256
        ne_it = ECAP // 16
        ne2_it_s = E2CAP // 16

        def h2_body(i, _):
            key = ek_v[pl.ds(i * 16, 16)]
            valid = (i * 16 + iota) < ce
            bin2 = lax.shift_right_logical(key, 13) & jnp.int32(0xFF)
            plsc.addupdate_scatter(lh2_v, [bin2 + lanemul2], ones16,
                                   mask=valid)
            return 0
        lax.fori_loop(0, ne_it, h2_body, 0)

        def merge2(j, _):
            acc = zeros16
            for l in range(16):
                acc = acc + lh2_v[pl.ds(l * 256 + j * 16, 16)]
            mh2_v[pl.ds(j * 16, 16)] = acc
            return 0
        lax.fori_loop(0, 16, merge2, 0)

        def cut2_body(jj, carry):
            c_above, cmax2 = carry
            j = 15 - jj
            chunk = mh2_v[pl.ds(j * 16, 16)]
            csum = plsc.cumsum(chunk)
            tot = jnp.sum(chunk)
            cnt = c_above + (tot - csum + chunk)
            bins = iota + j * 16
            cand = jnp.where(cnt >= need1, bins, jnp.int32(-1))
            return (c_above + tot, jnp.maximum(cmax2, cand))
        _, cmax2 = lax.fori_loop(
            0, 16, cut2_body, (jnp.int32(0), jnp.full((16,), -1, jnp.int32)))
        c2 = jnp.max(cmax2)

        def ge2_body(j, acc):
            chunk = mh2_v[pl.ds(j * 16, 16)]
            bins = iota + j * 16
            return acc + jnp.where(bins > c2, chunk, zeros16)
        g2_acc = lax.fori_loop(0, 16, ge2_body, zeros16)
        need2 = need1 - jnp.sum(g2_acc)

        def cls2(i, carry):
            cw, ce2 = carry
            key = ek_v[pl.ds(i * 16, 16)]
            gidx = ei_v[pl.ds(i * 16, 16)]
            valid = (i * 16 + iota) < ce
            bin2 = lax.shift_right_logical(key, 13) & jnp.int32(0xFF)
            cw = _append(wk_v, wi_v, cw, WCAP, key, gidx,
                         valid & (bin2 > c2))
            ce2 = _append(e2k_v, e2i_v, ce2, E2CAP, key, gidx,
                          valid & (bin2 == c2))
            return (cw, ce2)
        cw, ce2 = lax.fori_loop(0, ne_it, cls2, (cw, jnp.int32(0)))

        k0 = e2k_v[pl.ds(0, 16)]
        hb = _lane0(k0) & jnp.int32(~0x1FFF)

        def vb_round(r, t):
            candk = t | lax.shift_left(jnp.int32(1), jnp.int32(12) - r)
            acc = zeros16
            for i in range(E2CAP // 16):
                key = e2k_v[pl.ds(i * 16, 16)]
                valid = (i * 16 + iota) < ce2
                acc = acc + jnp.where(valid & (key >= candk), ones16, zeros16)
            return jnp.where(jnp.sum(acc) >= need2, candk, t)
        tkey = lax.fori_loop(0, 13, vb_round, hb)

        gacc = zeros16
        for i in range(E2CAP // 16):
            key = e2k_v[pl.ds(i * 16, 16)]
            valid = (i * 16 + iota) < ce2
            gacc = gacc + jnp.where(valid & (key > tkey), ones16, zeros16)
        need3 = need2 - jnp.sum(gacc)

        def ib_round(r, t2):
            candi = t2 | lax.shift_left(jnp.int32(1), jnp.int32(19) - r)
            acc = zeros16
            for i in range(E2CAP // 16):
                key = e2k_v[pl.ds(i * 16, 16)]
                gidx = e2i_v[pl.ds(i * 16, 16)]
                valid = (i * 16 + iota) < ce2
                m = valid & (key == tkey) & (gidx < candi)
                acc = acc + jnp.where(m, ones16, zeros16)
            return jnp.where(jnp.sum(acc) < need3, candi, t2)
        t2 = lax.fori_loop(0, 20, ib_round, jnp.int32(0))

        for i in range(E2CAP // 16):
            key = e2k_v[pl.ds(i * 16, 16)]
            gidx = e2i_v[pl.ds(i * 16, 16)]
            valid = (i * 16 + iota) < ce2
            m = valid & ((key > tkey) | ((key == tkey) & (gidx <= t2)))
            cw = _append(wk_v, wi_v, cw, WCAP, key, gidx, m)

        lo = c * HALF
        for t in range(32):
            key = wk_v[pl.ds(t * 16, 16)]
            gidx = wi_v[pl.ds(t * 16, 16)]
            own = (gidx >= lo) & (gidx < lo + HALF)
            si_v[t // 8, pl.ds((t % 8) * 16, 16)] = jnp.where(
                own, gidx, jnp.full((16,), NP, jnp.int32))
            sv_v[t // 8, pl.ds((t % 8) * 16, 16)] = jnp.where(
                own, _val_of(key), zeros16f)

    plsc.subcore_barrier()

    @pl.when(s == 0)
    def _scatter():
        for j in range(4):
            pltpu.async_copy(sv_v.at[j], out_hbm.at[si_v.at[j]], sem).wait()


def _make_stage1():
  return functools.partial(
    pl.kernel,
    mesh=_get_mesh(),
    name="sc_stage1_hist_compact",
    compiler_params=pltpu.CompilerParams(needs_layout_passes=False),
    out_type=[
        jax.ShapeDtypeStruct((NCAND,), jnp.int32),
        jax.ShapeDtypeStruct((NCAND,), jnp.int32),
        jax.ShapeDtypeStruct((2, NB), jnp.int32),
    ],
    scratch_types=[
        pltpu.VMEM((SL,), jnp.float32),
        pltpu.VMEM((NB * 16,), jnp.int32),
        pltpu.VMEM((NB,), jnp.int32),
        pltpu.VMEM((16, NB), jnp.int32),
        pltpu.VMEM((CAP,), jnp.int32),
        pltpu.VMEM((CAP,), jnp.int32),
        pltpu.VMEM((16,), jnp.int32),
        pltpu.VMEM_SHARED((16, NB), jnp.int32),
        pltpu.VMEM_SHARED((16,), jnp.int32),
    ]
  )(_stage1)


def _make_stage2():
  return functools.partial(
    pl.kernel,
    mesh=_get_mesh(),
    name="sc_stage2_select_scatter",
    compiler_params=pltpu.CompilerParams(needs_layout_passes=False),
    out_type=[jax.ShapeDtypeStruct((NPP,), jnp.float32)],
    scratch_types=[
        pltpu.VMEM((ZCH,), jnp.float32),
        pltpu.VMEM((NCAND,), jnp.int32),
        pltpu.VMEM((NCAND,), jnp.int32),
        pltpu.VMEM((2, NB), jnp.int32),
        pltpu.VMEM((NB,), jnp.int32),
        pltpu.VMEM((WCAP,), jnp.int32),
        pltpu.VMEM((WCAP,), jnp.int32),
        pltpu.VMEM((ECAP,), jnp.int32),
        pltpu.VMEM((ECAP,), jnp.int32),
        pltpu.VMEM((256 * 16,), jnp.int32),
        pltpu.VMEM((256,), jnp.int32),
        pltpu.VMEM((E2CAP,), jnp.int32),
        pltpu.VMEM((E2CAP,), jnp.int32),
        pltpu.VMEM((4, 128), jnp.int32),
        pltpu.VMEM((4, 128), jnp.float32),
        pltpu.SemaphoreType.DMA,
    ],
  )(_stage2)


@jax.jit
def kernel(x):
    xp = jnp.concatenate(
        [x, jnp.full((NP - NN,), -jnp.inf, dtype=jnp.float32)])
    ck, ci, h2 = _make_stage1()(xp)
    (out,) = _make_stage2()(ck, ci, h2)
    return out[:NN]


# speedup vs baseline: 2.0044x; 1.9540x over previous
import functools

import jax
import jax.numpy as jnp
from jax import lax
from jax.experimental import pallas as pl
from jax.experimental.pallas import tpu as pltpu
from jax.experimental.pallas import tpu_sc as plsc

NK = 512
NN = 1_000_000
SL = 31_264
NP = 32 * SL
NB = 2048
CAP = 128
NCAND = 32 * CAP
ITER = SL // 16

SL2 = 31_280
NPP = 32 * SL2
ZCH = SL2 // 5
HALF = 16 * SL2

WCAP = 544
ECAP = 2048
E2CAP = 128

IMIN = -2147483648

_MESH_CACHE = []


def _get_mesh():
    if not _MESH_CACHE:
        _MESH_CACHE.append(plsc.VectorSubcoreMesh(
            core_axis_name="c", subcore_axis_name="s"))
    return _MESH_CACHE[0]


def _key_of(v16):
    b = lax.bitcast_convert_type(v16, jnp.int32)
    sgn = lax.shift_right_arithmetic(b, 31)
    return b ^ (sgn & jnp.int32(0x7FFFFFFF))


def _val_of(k16):
    sgn = lax.shift_right_arithmetic(k16, 31)
    return lax.bitcast_convert_type(k16 ^ (sgn & jnp.int32(0x7FFFFFFF)), jnp.float32)


def _iota16():
    return lax.broadcasted_iota(jnp.int32, (16,), 0)


def _lane0(v):
    return lax.squeeze(lax.slice(v, (0,), (1,)), (0,))


def _lane15(v):
    return lax.squeeze(lax.slice(v, (15,), (16,)), (0,))


def _stage1(x_hbm, ck_hbm, ci_hbm, h2_hbm,
            data_v, lhist_v, mhist_v, ghist_v, ck_v, ci_v, cut_v, sh_v, shc_v):
    c = lax.axis_index("c")
    s = lax.axis_index("s")
    wid = c * 16 + s
    base = wid * SL
    iota = _iota16()
    zeros16 = jnp.zeros((16,), jnp.int32)
    ones16 = jnp.ones((16,), jnp.int32)

    pltpu.sync_copy(x_hbm.at[pl.ds(base, SL)], data_v)

    def zl(i, _):
        lhist_v[pl.ds(i * 16, 16)] = zeros16
        return 0
    lax.fori_loop(0, NB * 16 // 16, zl, 0, unroll=8)

    lanemul = iota * NB

    def hist_body2(i):
        v = data_v[pl.ds(i * 16, 16)]
        key = _key_of(v)
        bin1 = lax.shift_right_arithmetic(key, 21) + jnp.int32(1024)
        plsc.addupdate_scatter(lhist_v, [bin1 + lanemul], ones16)
    plsc.parallel_loop(0, ITER, 1, unroll=8)(hist_body2)

    def merge_body(j, _):
        acc = zeros16
        for l in range(16):
            acc = acc + lhist_v[pl.ds(l * NB + j * 16, 16)]
        mhist_v[pl.ds(j * 16, 16)] = acc
        return 0
    lax.fori_loop(0, NB // 16, merge_body, 0)

    pltpu.sync_copy(mhist_v, sh_v.at[s])
    plsc.subcore_barrier()

    @pl.when(s == 0)
    def _select_cutoff():
        pltpu.sync_copy(sh_v, ghist_v)

        def merge2(j, _):
            acc = zeros16
            for l in range(16):
                acc = acc + ghist_v[l, pl.ds(j * 16, 16)]
            mhist_v[pl.ds(j * 16, 16)] = acc
            return 0
        lax.fori_loop(0, NB // 16, merge2, 0)

        pltpu.sync_copy(mhist_v, h2_hbm.at[c])

        def cut_body(jj, carry):
            c_above, cmax = carry
            j = NB // 16 - 1 - jj
            chunk = mhist_v[pl.ds(j * 16, 16)]
            csum = plsc.cumsum(chunk)
            tot = _lane15(csum)
            suffix = tot - csum + chunk
            cnt = c_above + suffix
            bins = iota + j * 16
            cand = jnp.where(cnt >= NK, bins, jnp.int32(-1))
            return (c_above + tot, jnp.maximum(cmax, cand))
        _, cmax = lax.fori_loop(
            0, NB // 16, cut_body,
            (jnp.int32(0), jnp.full((16,), -1, jnp.int32)))
        cbin = jnp.max(cmax)
        cut_key = lax.shift_left(cbin - jnp.int32(1024), 21)
        cut_v[...] = jnp.full((16,), 1, jnp.int32) * cut_key
        pltpu.sync_copy(cut_v, shc_v)

    plsc.subcore_barrier()
    pltpu.sync_copy(shc_v, cut_v)
    cut_key = jnp.max(cut_v[...])

    minkey16 = jnp.full((16,), IMIN, jnp.int32)
    pad16 = jnp.full((16,), NP, jnp.int32)
    for t in range(CAP // 16):
        ck_v[pl.ds(t * 16, 16)] = minkey16
        ci_v[pl.ds(t * 16, 16)] = pad16

    def cp_body(i, cnt):
        v = data_v[pl.ds(i * 16, 16)]
        key = _key_of(v)
        m = key >= cut_key
        gidx = base + i * 16 + iota
        cc = jnp.minimum(cnt, CAP - 16)
        plsc.store_compressed(ck_v.at[pl.ds(cc, 16)], key, mask=m)
        plsc.store_compressed(ci_v.at[pl.ds(cc, 16)], gidx, mask=m)
        return cnt + _lane0(plsc.all_reduce_population_count(m))
    lax.fori_loop(0, ITER, cp_body, jnp.int32(0), unroll=8)

    pltpu.sync_copy(ck_v, ck_hbm.at[pl.ds(wid * CAP, CAP)])
    pltpu.sync_copy(ci_v, ci_hbm.at[pl.ds(wid * CAP, CAP)])


def _append(dst_k, dst_i, cnt, cap, key, gidx, m):
    cc = jnp.minimum(cnt, cap - 16)
    plsc.store_compressed(dst_k.at[pl.ds(cc, 16)], key, mask=m)
    plsc.store_compressed(dst_i.at[pl.ds(cc, 16)], gidx, mask=m)
    return cnt + _lane0(plsc.all_reduce_population_count(m))


def _stage2(ck_hbm, ci_hbm, h2_hbm, out_hbm,
            slice_v, ckv, civ, h2v, mh_v, wk_v, wi_v, ek_v, ei_v,
            lh2_v, mh2_v, e2k_v, e2i_v, swk_v, swi_v):
    c = lax.axis_index("c")
    s = lax.axis_index("s")
    wid = c * 16 + s
    base2 = wid * SL2
    iota = _iota16()
    zeros16f = jnp.zeros((16,), jnp.float32)
    zeros16 = jnp.zeros((16,), jnp.int32)

    def zf(i, _):
        slice_v[pl.ds(i * 16, 16)] = zeros16f
        return 0
    lax.fori_loop(0, SL2 // 16, zf, 0, unroll=8)

    @pl.when(s == 0)
    def _select():
        pltpu.sync_copy(ck_hbm, ckv)
        pltpu.sync_copy(ci_hbm, civ)
        pltpu.sync_copy(h2_hbm, h2v)

        def mh_body(j, _):
            mh_v[pl.ds(j * 16, 16)] = (h2v[0, pl.ds(j * 16, 16)]
                                       + h2v[1, pl.ds(j * 16, 16)])
            return 0
        lax.fori_loop(0, NB // 16, mh_body, 0)

        def cut_body(jj, carry):
            c_above, cmax = carry
            j = NB // 16 - 1 - jj
            chunk = mh_v[pl.ds(j * 16, 16)]
            csum = plsc.cumsum(chunk)
            tot = _lane15(csum)
            cnt = c_above + (tot - csum + chunk)
            bins = iota + j * 16
            cand = jnp.where(cnt >= NK, bins, jnp.int32(-1))
            return (c_above + tot, jnp.maximum(cmax, cand))
        _, cmax = lax.fori_loop(
            0, NB // 16, cut_body,
            (jnp.int32(0), jnp.full((16,), -1, jnp.int32)))
        c1 = jnp.max(cmax)

        def ge_body(j, acc):
            g_acc, e_acc = acc
            chunk = mh_v[pl.ds(j * 16, 16)]
            bins = iota + j * 16
            g_acc = g_acc + jnp.where(bins > c1, chunk, zeros16)
            e_acc = e_acc + jnp.where(bins == c1, chunk, zeros16)
            return (g_acc, e_acc)
        g_acc, e_acc = lax.fori_loop(0, NB // 16, ge_body, (zeros16, zeros16))
        need1 = NK - jnp.sum(g_acc)

        minkey16 = jnp.full((16,), IMIN, jnp.int32)
        pad16 = jnp.full((16,), NP, jnp.int32)
        for t in range(WCAP // 16):
            wk_v[pl.ds(t * 16, 16)] = minkey16
            wi_v[pl.ds(t * 16, 16)] = pad16

        def cls1(i, carry):
            cw, ce = carry
            key = ckv[pl.ds(i * 16, 16)]
            gidx = civ[pl.ds(i * 16, 16)]
            bin1 = lax.shift_right_arithmetic(key, 21) + jnp.int32(1024)
            cw = _append(wk_v, wi_v, cw, WCAP, key, gidx, bin1 > c1)
            ce = _append(ek_v, ei_v, ce, ECAP, key, gidx, bin1 == c1)
            return (cw, ce)
        cw, ce = lax.fori_loop(0, NCAND // 16, cls1,
                               (jnp.int32(0), jnp.int32(0)), unroll=2)

        def zl2(i, _):
            lh2_v[pl.ds(i * 16, 16)] = zeros16
            return 0
        lax.fori_loop(0, 256 * 16 // 16, zl2, 0, unroll=8)

        ones16 = jnp.ones((16,), jnp.int32)
        lanemul2 = iota * 256

        def h2_body(i, _):
            key = ek_v[pl.ds(i * 16, 16)]
            valid = (i * 16 + iota) < ce
            bin2 = lax.shift_right_logical(key, 13) & jnp.int32(0xFF)
            plsc.addupdate_scatter(lh2_v, [bin2 + lanemul2], ones16,
                                   mask=valid)
            return 0
        lax.fori_loop(0, ECAP // 16, h2_body, 0)

        def merge2(j, _):
            acc = zeros16
            for l in range(16):
                acc = acc + lh2_v[pl.ds(l * 256 + j * 16, 16)]
            mh2_v[pl.ds(j * 16, 16)] = acc
            return 0
        lax.fori_loop(0, 16, merge2, 0)

        def cut2_body(jj, carry):
            c_above, cmax2 = carry
            j = 15 - jj
            chunk = mh2_v[pl.ds(j * 16, 16)]
            csum = plsc.cumsum(chunk)
            tot = _lane15(csum)
            cnt = c_above + (tot - csum + chunk)
            bins = iota + j * 16
            cand = jnp.where(cnt >= need1, bins, jnp.int32(-1))
            return (c_above + tot, jnp.maximum(cmax2, cand))
        _, cmax2 = lax.fori_loop(
            0, 16, cut2_body, (jnp.int32(0), jnp.full((16,), -1, jnp.int32)))
        c2 = jnp.max(cmax2)

        def ge2_body(j, acc):
            chunk = mh2_v[pl.ds(j * 16, 16)]
            bins = iota + j * 16
            return acc + jnp.where(bins > c2, chunk, zeros16)
        g2_acc = lax.fori_loop(0, 16, ge2_body, zeros16)
        need2 = need1 - jnp.sum(g2_acc)

        def cls2(i, carry):
            cw, ce2 = carry
            key = ek_v[pl.ds(i * 16, 16)]
            gidx = ei_v[pl.ds(i * 16, 16)]
            valid = (i * 16 + iota) < ce
            bin2 = lax.shift_right_logical(key, 13) & jnp.int32(0xFF)
            cw = _append(wk_v, wi_v, cw, WCAP, key, gidx,
                         valid & (bin2 > c2))
            ce2 = _append(e2k_v, e2i_v, ce2, E2CAP, key, gidx,
                          valid & (bin2 == c2))
            return (cw, ce2)
        cw, ce2 = lax.fori_loop(0, ECAP // 16, cls2, (cw, jnp.int32(0)))

        k0 = e2k_v[pl.ds(0, 16)]
        hb = _lane0(k0) & jnp.int32(~0x1FFF)

        def vb_round(r, t):
            candk = t | lax.shift_left(jnp.int32(1), jnp.int32(12) - r)
            acc = zeros16
            for i in range(E2CAP // 16):
                key = e2k_v[pl.ds(i * 16, 16)]
                valid = (i * 16 + iota) < ce2
                acc = acc + jnp.where(valid & (key >= candk), ones16, zeros16)
            return jnp.where(jnp.sum(acc) >= need2, candk, t)
        tkey = lax.fori_loop(0, 13, vb_round, hb)

        gacc = zeros16
        for i in range(E2CAP // 16):
            key = e2k_v[pl.ds(i * 16, 16)]
            valid = (i * 16 + iota) < ce2
            gacc = gacc + jnp.where(valid & (key > tkey), ones16, zeros16)
        need3 = need2 - jnp.sum(gacc)

        def ib_round(r, t2):
            candi = t2 | lax.shift_left(jnp.int32(1), jnp.int32(19) - r)
            acc = zeros16
            for i in range(E2CAP // 16):
                key = e2k_v[pl.ds(i * 16, 16)]
                gidx = e2i_v[pl.ds(i * 16, 16)]
                valid = (i * 16 + iota) < ce2
                m = valid & (key == tkey) & (gidx < candi)
                acc = acc + jnp.where(m, ones16, zeros16)
            return jnp.where(jnp.sum(acc) < need3, candi, t2)
        t2 = lax.fori_loop(0, 20, ib_round, jnp.int32(0))

        for i in range(E2CAP // 16):
            key = e2k_v[pl.ds(i * 16, 16)]
            gidx = e2i_v[pl.ds(i * 16, 16)]
            valid = (i * 16 + iota) < ce2
            m = valid & ((key > tkey) | ((key == tkey) & (gidx <= t2)))
            cw = _append(wk_v, wi_v, cw, WCAP, key, gidx, m)

        pltpu.sync_copy(wk_v, swk_v)
        pltpu.sync_copy(wi_v, swi_v)

    plsc.subcore_barrier()

    pltpu.sync_copy(swk_v, wk_v)
    pltpu.sync_copy(swi_v, wi_v)
    for t in range(WCAP // 16):
        key = wk_v[pl.ds(t * 16, 16)]
        gidx = wi_v[pl.ds(t * 16, 16)]
        own = (gidx >= base2) & (gidx < base2 + SL2)
        local = gidx - base2
        plsc.store_scatter(slice_v, [local], _val_of(key), mask=own)

    pltpu.sync_copy(slice_v, out_hbm.at[pl.ds(base2, SL2)])


def _make_stage1():
  return functools.partial(
    pl.kernel,
    mesh=_get_mesh(),
    name="sc_stage1_hist_compact",
    compiler_params=pltpu.CompilerParams(needs_layout_passes=False),
    out_type=[
        jax.ShapeDtypeStruct((NCAND,), jnp.int32),
        jax.ShapeDtypeStruct((NCAND,), jnp.int32),
        jax.ShapeDtypeStruct((2, NB), jnp.int32),
    ],
    scratch_types=[
        pltpu.VMEM((SL,), jnp.float32),
        pltpu.VMEM((NB * 16,), jnp.int32),
        pltpu.VMEM((NB,), jnp.int32),
        pltpu.VMEM((16, NB), jnp.int32),
        pltpu.VMEM((CAP,), jnp.int32),
        pltpu.VMEM((CAP,), jnp.int32),
        pltpu.VMEM((16,), jnp.int32),
        pltpu.VMEM_SHARED((16, NB), jnp.int32),
        pltpu.VMEM_SHARED((16,), jnp.int32),
    ]
  )(_stage1)


def _make_stage2():
  return functools.partial(
    pl.kernel,
    mesh=_get_mesh(),
    name="sc_stage2_select_scatter",
    compiler_params=pltpu.CompilerParams(needs_layout_passes=False),
    out_type=[jax.ShapeDtypeStruct((NPP,), jnp.float32)],
    scratch_types=[
        pltpu.VMEM((SL2,), jnp.float32),
        pltpu.VMEM((NCAND,), jnp.int32),
        pltpu.VMEM((NCAND,), jnp.int32),
        pltpu.VMEM((2, NB), jnp.int32),
        pltpu.VMEM((NB,), jnp.int32),
        pltpu.VMEM((WCAP,), jnp.int32),
        pltpu.VMEM((WCAP,), jnp.int32),
        pltpu.VMEM((ECAP,), jnp.int32),
        pltpu.VMEM((ECAP,), jnp.int32),
        pltpu.VMEM((256 * 16,), jnp.int32),
        pltpu.VMEM((256,), jnp.int32),
        pltpu.VMEM((E2CAP,), jnp.int32),
        pltpu.VMEM((E2CAP,), jnp.int32),
        pltpu.VMEM_SHARED((WCAP,), jnp.int32),
        pltpu.VMEM_SHARED((WCAP,), jnp.int32),
    ],
  )(_stage2)


@jax.jit
def kernel(x):
    xp = jnp.concatenate(
        [x, jnp.full((NP - NN,), -jnp.inf, dtype=jnp.float32)])
    ck, ci, h2 = _make_stage1()(xp)
    (out,) = _make_stage2()(ck, ci, h2)
    return out[:NN]
